# baseline (speedup 1.0000x reference)
import functools
import math

import jax
import jax.numpy as jnp
from jax import lax
from jax.experimental import pallas as pl
from jax.experimental.pallas import tpu as pltpu
from jax.experimental.pallas import tpu_sc as plsc

D = 16
SCALE = math.sqrt(D)
NC, NS = 2, 16
NW = NC * NS
STREAM = 128
CHUNK_STREAMS = 10
CHUNK = STREAM * CHUNK_STREAMS


def _emb_body(n_chunks, W_hbm, idx_hbm, out_hbm, idx_v, rows_v, sem):
    wid = lax.axis_index("s") * NC + lax.axis_index("c")
    rows_per_w = n_chunks * CHUNK
    idx_rows = n_chunks * CHUNK_STREAMS
    base = wid * rows_per_w

    pltpu.sync_copy(idx_hbm.at[pl.ds(wid * idx_rows, idx_rows)], idx_v)

    def fire(chunk_i, region):
        for j in range(CHUNK_STREAMS):
            pltpu.async_copy(
                W_hbm.at[idx_v.at[chunk_i * CHUNK_STREAMS + j]],
                rows_v.at[pl.ds(region * CHUNK + j * STREAM, STREAM)],
                sem,
            )

    def drain(region):
        pltpu.make_async_copy(
            W_hbm.at[pl.ds(0, CHUNK)],
            rows_v.at[pl.ds(region * CHUNK, CHUNK)],
            sem,
        ).wait()

    fire(0, 0)

    def step(g, _):
        boff = (g % 2) * CHUNK
        drain(g % 2)

        @pl.when(g + 1 < n_chunks)
        def _():
            for j in range(CHUNK_STREAMS):
                pltpu.async_copy(
                    W_hbm.at[idx_v.at[(g + 1) * CHUNK_STREAMS + j]],
                    rows_v.at[pl.ds(((g + 1) % 2) * CHUNK + j * STREAM, STREAM)],
                    sem,
                )

        def scale(i, _):
            rows_v[boff + i] = rows_v[boff + i] * SCALE
            return 0

        lax.fori_loop(0, CHUNK, scale, 0, unroll=4)
        pltpu.sync_copy(
            rows_v.at[pl.ds(boff, CHUNK)],
            out_hbm.at[pl.ds(base + g * CHUNK, CHUNK)],
        )
        return 0

    lax.fori_loop(0, n_chunks, step, 0)


@functools.partial(jax.jit, static_argnames=("n_rows",))
def _emb_lookup(x_flat2d, W, n_rows):
    n_chunks = n_rows // (NW * CHUNK)
    idx_rows = n_chunks * CHUNK_STREAMS
    mesh = plsc.VectorSubcoreMesh(core_axis_name="c", subcore_axis_name="s")
    run = pl.kernel(
        functools.partial(_emb_body, n_chunks),
        out_type=jax.ShapeDtypeStruct((n_rows, D), jnp.float32),
        mesh=mesh,
        scratch_types=[
            pltpu.VMEM((NW // NW * idx_rows, STREAM), jnp.int32),
            pltpu.VMEM((2 * CHUNK, D), jnp.float32),
            pltpu.SemaphoreType.DMA,
        ],
        compiler_params=pltpu.CompilerParams(use_tc_tiling_on_sc=False),
    )
    return run(W, x_flat2d)


def kernel(x, W):
    b, h = x.shape
    n_rows = b * h
    x_flat2d = x.reshape(n_rows // STREAM, STREAM)
    out = _emb_lookup(x_flat2d, W, n_rows)
    return out.reshape(b, h, D)


# speedup vs baseline: 1.1982x; 1.1982x over previous
import functools
import math

import jax
import jax.numpy as jnp
from jax import lax
from jax.experimental import pallas as pl
from jax.experimental.pallas import tpu as pltpu
from jax.experimental.pallas import tpu_sc as plsc

D = 16
SCALE = math.sqrt(D)
NC, NS = 2, 16
NW = NC * NS
STREAM = 128
TASK = 1024
TASK_STREAMS = TASK // STREAM


def _emb_body(n_tasks_per_w, n_batch, W_hbm, idx_hbm, out_hbm,
              idx_v, gath_v, tbuf_v, gsem, wsem):
    wid = lax.axis_index("s") * NC + lax.axis_index("c")
    idx_rows = n_tasks_per_w * TASK_STREAMS
    qs_per_h = n_batch // TASK

    pltpu.sync_copy(idx_hbm.at[pl.ds(wid * idx_rows, idx_rows)], idx_v)

    lane = lax.broadcasted_iota(jnp.int32, (D,), 0)

    def fire(g, buf):
        for j in range(TASK_STREAMS):
            pltpu.async_copy(
                W_hbm.at[idx_v.at[g * TASK_STREAMS + j]],
                gath_v.at[buf, pl.ds(j * STREAM, STREAM)],
                gsem,
            )

    fire(0, 0)

    def step(g, _):
        cb = g % 2
        t = wid * n_tasks_per_w + g
        h = t // qs_per_h
        q = t % qs_per_h

        pltpu.make_async_copy(
            W_hbm.at[pl.ds(0, TASK)], gath_v.at[cb], gsem
        ).wait()

        @pl.when(g + 1 < n_tasks_per_w)
        def _():
            for j in range(TASK_STREAMS):
                pltpu.async_copy(
                    W_hbm.at[idx_v.at[(g + 1) * TASK_STREAMS + j]],
                    gath_v.at[(g + 1) % 2, pl.ds(j * STREAM, STREAM)],
                    gsem,
                )

        @pl.when(g >= 2)
        def _():
            pltpu.make_async_copy(
                W_hbm.at[pl.ds(0, D)], tbuf_v.at[cb], wsem
            ).wait()

        buf_ix = jnp.full((D,), cb, jnp.int32)

        def transpose_scale(b, _):
            row = gath_v[cb, b] * SCALE
            plsc.store_scatter(
                tbuf_v, [buf_ix, lane, jnp.full((D,), b, jnp.int32)], row
            )
            return 0

        lax.fori_loop(0, TASK, transpose_scale, 0, unroll=8)

        pltpu.async_copy(
            tbuf_v.at[cb],
            out_hbm.at[pl.ds(h * D, D), pl.ds(q * TASK, TASK)],
            wsem,
        )
        return 0

    lax.fori_loop(0, n_tasks_per_w, step, 0)

    pltpu.make_async_copy(W_hbm.at[pl.ds(0, D)], tbuf_v.at[0], wsem).wait()
    pltpu.make_async_copy(W_hbm.at[pl.ds(0, D)], tbuf_v.at[1], wsem).wait()


@functools.partial(jax.jit, static_argnames=("n_hist", "n_batch"))
def _emb_lookup(x_t2d, W, n_hist, n_batch):
    n_rows = n_hist * n_batch
    n_tasks_per_w = n_rows // (NW * TASK)
    idx_rows = n_tasks_per_w * TASK_STREAMS
    mesh = plsc.VectorSubcoreMesh(core_axis_name="c", subcore_axis_name="s")
    run = pl.kernel(
        functools.partial(_emb_body, n_tasks_per_w, n_batch),
        out_type=jax.ShapeDtypeStruct((n_hist * D, n_batch), jnp.float32),
        mesh=mesh,
        scratch_types=[
            pltpu.VMEM((idx_rows, STREAM), jnp.int32),
            pltpu.VMEM((2, TASK, D), jnp.float32),
            pltpu.VMEM((2, D, TASK), jnp.float32),
            pltpu.SemaphoreType.DMA,
            pltpu.SemaphoreType.DMA,
        ],
        compiler_params=pltpu.CompilerParams(
            use_tc_tiling_on_sc=False, needs_layout_passes=False
        ),
    )
    return run(W, x_t2d)


def kernel(x, W):
    b, h = x.shape
    x_t2d = x.T.reshape((h * b) // STREAM, STREAM)
    out = _emb_lookup(x_t2d, W, h, b)
    return out.reshape(h, D, b).transpose(2, 0, 1)


# speedup vs baseline: 1.6520x; 1.3787x over previous
import functools
import math

import jax
import jax.numpy as jnp
from jax import lax
from jax.experimental import pallas as pl
from jax.experimental.pallas import tpu as pltpu
from jax.experimental.pallas import tpu_sc as plsc

D = 16
SCALE = math.sqrt(D)
NC, NS = 2, 16
NW = NC * NS
STREAM = 128
TASK = 1024
TASK_STREAMS = TASK // STREAM


def _emb_body(n_tasks_per_w, n_batch, W_hbm, idx_hbm, out_hbm,
              idx_v, gath_v, tbuf_v, gsem, wsem):
    wid = lax.axis_index("s") * NC + lax.axis_index("c")
    idx_rows = n_tasks_per_w * TASK_STREAMS
    qs_per_h = n_batch // TASK

    pltpu.sync_copy(idx_hbm.at[pl.ds(wid * idx_rows, idx_rows)], idx_v)

    lane = lax.broadcasted_iota(jnp.int32, (D,), 0)

    def fire(g, buf):
        for j in range(TASK_STREAMS):
            pltpu.async_copy(
                W_hbm.at[idx_v.at[g * TASK_STREAMS + j]],
                gath_v.at[buf, pl.ds(j * STREAM, STREAM)],
                gsem,
            )

    fire(0, 0)

    def step(g, _):
        cb = g % 2
        t = wid * n_tasks_per_w + g
        h = t // qs_per_h
        q = t % qs_per_h

        pltpu.make_async_copy(
            W_hbm.at[pl.ds(0, TASK)], gath_v.at[cb], gsem
        ).wait()

        @pl.when(g + 1 < n_tasks_per_w)
        def _():
            for j in range(TASK_STREAMS):
                pltpu.async_copy(
                    W_hbm.at[idx_v.at[(g + 1) * TASK_STREAMS + j]],
                    gath_v.at[(g + 1) % 2, pl.ds(j * STREAM, STREAM)],
                    gsem,
                )

        @pl.when(g >= 2)
        def _():
            pltpu.make_async_copy(
                W_hbm.at[pl.ds(0, D)], tbuf_v.at[cb], wsem
            ).wait()

        buf_ix = jnp.full((D,), cb, jnp.int32)

        @functools.partial(
            plsc.parallel_loop, 0, TASK, unroll=8,
            carry=jnp.zeros((D,), jnp.int32),
        )
        def _(b, bvec):
            row = gath_v[cb, b] * SCALE
            plsc.store_scatter(tbuf_v, [buf_ix, lane, bvec], row)
            return bvec + 1

        pltpu.async_copy(
            tbuf_v.at[cb],
            out_hbm.at[pl.ds(h * D, D), pl.ds(q * TASK, TASK)],
            wsem,
        )
        return 0

    lax.fori_loop(0, n_tasks_per_w, step, 0)

    pltpu.make_async_copy(W_hbm.at[pl.ds(0, D)], tbuf_v.at[0], wsem).wait()
    pltpu.make_async_copy(W_hbm.at[pl.ds(0, D)], tbuf_v.at[1], wsem).wait()


@functools.partial(jax.jit, static_argnames=("n_hist", "n_batch"))
def _emb_lookup(x_t2d, W, n_hist, n_batch):
    n_rows = n_hist * n_batch
    n_tasks_per_w = n_rows // (NW * TASK)
    idx_rows = n_tasks_per_w * TASK_STREAMS
    mesh = plsc.VectorSubcoreMesh(core_axis_name="c", subcore_axis_name="s")
    run = pl.kernel(
        functools.partial(_emb_body, n_tasks_per_w, n_batch),
        out_type=jax.ShapeDtypeStruct((n_hist * D, n_batch), jnp.float32),
        mesh=mesh,
        scratch_types=[
            pltpu.VMEM((idx_rows, STREAM), jnp.int32),
            pltpu.VMEM((2, TASK, D), jnp.float32),
            pltpu.VMEM((2, D, TASK), jnp.float32),
            pltpu.SemaphoreType.DMA,
            pltpu.SemaphoreType.DMA,
        ],
        compiler_params=pltpu.CompilerParams(
            use_tc_tiling_on_sc=False, needs_layout_passes=False
        ),
    )
    return run(W, x_t2d)


def kernel(x, W):
    b, h = x.shape
    x_t2d = x.T.reshape((h * b) // STREAM, STREAM)
    out = _emb_lookup(x_t2d, W, h, b)
    return out.reshape(h, D, b).transpose(2, 0, 1)


# speedup vs baseline: 2.1425x; 1.2969x over previous
import functools
import math

import jax
import jax.numpy as jnp
from jax import lax
from jax.experimental import pallas as pl
from jax.experimental.pallas import tpu as pltpu
from jax.experimental.pallas import tpu_sc as plsc

D = 16
SCALE = math.sqrt(D)
NC, NS = 2, 16
NW = NC * NS
STREAM = 128
TASK = 1024
TASK_STREAMS = TASK // STREAM


def _tc_transpose_body(bc, wt_ref, out_ref):
    t = wt_ref[...].T
    t8 = t.reshape(bc // 8, 8, 16)
    out_ref[...] = jnp.concatenate([t8[:, k, :] for k in range(8)], axis=1)


def _tc_transpose(Wt, v):
    bc = 8192
    return pl.pallas_call(
        functools.partial(_tc_transpose_body, bc),
        grid=(pl.cdiv(v, bc),),
        in_specs=[pl.BlockSpec((D, bc), lambda i: (0, i))],
        out_specs=pl.BlockSpec((bc // 8, 128), lambda i: (i, 0)),
        out_shape=jax.ShapeDtypeStruct((v * D // 128, 128), jnp.float32),
    )(Wt)


def _emb_body(n_tasks_per_w, n_batch, W_hbm, idx_hbm, out_hbm,
              idx_v, gath_v, tbuf_v, gsem, wsem):
    wid = lax.axis_index("s") * NC + lax.axis_index("c")
    idx_rows = n_tasks_per_w * TASK_STREAMS
    qs_per_h = n_batch // TASK

    pltpu.sync_copy(idx_hbm.at[pl.ds(wid * idx_rows, idx_rows)], idx_v)

    lane = lax.broadcasted_iota(jnp.int32, (D,), 0)

    def fire(g, buf):
        for j in range(TASK_STREAMS):
            pltpu.async_copy(
                W_hbm.at[idx_v.at[g * TASK_STREAMS + j]],
                gath_v.at[buf, pl.ds(j * STREAM, STREAM)],
                gsem,
            )

    fire(0, 0)

    def step(g, _):
        cb = g % 2
        t = wid * n_tasks_per_w + g
        h = t // qs_per_h
        q = t % qs_per_h

        pltpu.make_async_copy(
            W_hbm.at[pl.ds(0, TASK)], gath_v.at[cb], gsem
        ).wait()

        @pl.when(g + 1 < n_tasks_per_w)
        def _():
            for j in range(TASK_STREAMS):
                pltpu.async_copy(
                    W_hbm.at[idx_v.at[(g + 1) * TASK_STREAMS + j]],
                    gath_v.at[(g + 1) % 2, pl.ds(j * STREAM, STREAM)],
                    gsem,
                )

        @pl.when(g >= 2)
        def _():
            pltpu.make_async_copy(
                W_hbm.at[pl.ds(0, D)], tbuf_v.at[cb], wsem
            ).wait()

        buf_ix = jnp.full((D,), cb, jnp.int32)

        @functools.partial(
            plsc.parallel_loop, 0, TASK, unroll=8,
            carry=jnp.zeros((D,), jnp.int32),
        )
        def _(b, bvec):
            row = gath_v[cb, b] * SCALE
            plsc.store_scatter(tbuf_v, [buf_ix, lane, bvec], row)
            return bvec + 1

        pltpu.async_copy(
            tbuf_v.at[cb],
            out_hbm.at[pl.ds(h * D, D), pl.ds(q * TASK, TASK)],
            wsem,
        )
        return 0

    lax.fori_loop(0, n_tasks_per_w, step, 0)

    pltpu.make_async_copy(W_hbm.at[pl.ds(0, D)], tbuf_v.at[0], wsem).wait()
    pltpu.make_async_copy(W_hbm.at[pl.ds(0, D)], tbuf_v.at[1], wsem).wait()


@functools.partial(jax.jit, static_argnames=("n_hist", "n_batch"))
def _emb_lookup(x_t2d, W, n_hist, n_batch):
    n_rows = n_hist * n_batch
    n_tasks_per_w = n_rows // (NW * TASK)
    idx_rows = n_tasks_per_w * TASK_STREAMS
    mesh = plsc.VectorSubcoreMesh(core_axis_name="c", subcore_axis_name="s")
    run = pl.kernel(
        functools.partial(_emb_body, n_tasks_per_w, n_batch),
        out_type=jax.ShapeDtypeStruct((n_hist * D, n_batch), jnp.float32),
        mesh=mesh,
        scratch_types=[
            pltpu.VMEM((idx_rows, STREAM), jnp.int32),
            pltpu.VMEM((2, TASK, D), jnp.float32),
            pltpu.VMEM((2, D, TASK), jnp.float32),
            pltpu.SemaphoreType.DMA,
            pltpu.SemaphoreType.DMA,
        ],
        compiler_params=pltpu.CompilerParams(
            use_tc_tiling_on_sc=False, needs_layout_passes=False
        ),
    )
    return run(W, x_t2d)


def kernel(x, W):
    b, h = x.shape
    v = W.shape[0]
    x_t2d = x.T.reshape((h * b) // STREAM, STREAM)
    W_lin = _tc_transpose(W.T, v).reshape(v, D)
    out = _emb_lookup(x_t2d, W_lin, h, b)
    return out.reshape(h, D, b).transpose(2, 0, 1)


# speedup vs baseline: 4.0266x; 1.8793x over previous
import functools
import math

import jax
import jax.numpy as jnp
from jax import lax
from jax.experimental import pallas as pl
from jax.experimental.pallas import tpu as pltpu
from jax.experimental.pallas import tpu_sc as plsc

D = 16
SCALE = math.sqrt(D)
NC, NS = 2, 16
NW = NC * NS
STREAM = 128
TASK = 1024
TASK_STREAMS = TASK // STREAM


def _tc_transpose_body(bc, wt_ref, out_ref):
    cw = bc // 8
    u = jnp.concatenate([wt_ref[:, pl.ds(m * cw, cw)] for m in range(8)], axis=0)
    out_ref[...] = u.T


def _tc_transpose(Wt, v):
    bc = 8192
    nblk = pl.cdiv(v, bc)
    return pl.pallas_call(
        functools.partial(_tc_transpose_body, bc),
        grid=(nblk,),
        in_specs=[pl.BlockSpec((D, bc), lambda i: (0, i))],
        out_specs=pl.BlockSpec((bc // 8, 128), lambda i: (i, 0)),
        out_shape=jax.ShapeDtypeStruct((nblk * bc // 8, 128), jnp.float32),
    )(Wt)


def _emb_body(n_tasks_per_w, n_batch, W_hbm, idx_hbm, out_hbm,
              idx_v, gath_v, tbuf_v, gsem, wsem):
    wid = lax.axis_index("s") * NC + lax.axis_index("c")
    idx_rows = n_tasks_per_w * TASK_STREAMS
    qs_per_h = n_batch // TASK

    pltpu.sync_copy(idx_hbm.at[pl.ds(wid * idx_rows, idx_rows)], idx_v)

    @functools.partial(plsc.parallel_loop, 0, idx_rows, unroll=2)
    def _(r):
        for j in range(8):
            vv = idx_v[r, pl.ds(j * D, D)]
            idx_v[r, pl.ds(j * D, D)] = (
                (vv & -8192) + ((vv & 1023) << 3) + ((vv >> 10) & 7)
            )

    lane = lax.broadcasted_iota(jnp.int32, (D,), 0)

    def fire(g, buf):
        for j in range(TASK_STREAMS):
            pltpu.async_copy(
                W_hbm.at[idx_v.at[g * TASK_STREAMS + j]],
                gath_v.at[buf, pl.ds(j * STREAM, STREAM)],
                gsem,
            )

    fire(0, 0)

    def step(g, _):
        cb = g % 2
        t = wid * n_tasks_per_w + g
        h = t // qs_per_h
        q = t % qs_per_h

        pltpu.make_async_copy(
            W_hbm.at[pl.ds(0, TASK)], gath_v.at[cb], gsem
        ).wait()

        @pl.when(g + 1 < n_tasks_per_w)
        def _():
            for j in range(TASK_STREAMS):
                pltpu.async_copy(
                    W_hbm.at[idx_v.at[(g + 1) * TASK_STREAMS + j]],
                    gath_v.at[(g + 1) % 2, pl.ds(j * STREAM, STREAM)],
                    gsem,
                )

        @pl.when(g >= 2)
        def _():
            pltpu.make_async_copy(
                W_hbm.at[pl.ds(0, D)], tbuf_v.at[cb], wsem
            ).wait()

        buf_ix = jnp.full((D,), cb, jnp.int32)

        @functools.partial(
            plsc.parallel_loop, 0, TASK, unroll=8,
            carry=jnp.zeros((D,), jnp.int32),
        )
        def _(b, bvec):
            row = gath_v[cb, b] * SCALE
            plsc.store_scatter(tbuf_v, [buf_ix, lane, bvec], row)
            return bvec + 1

        pltpu.async_copy(
            tbuf_v.at[cb],
            out_hbm.at[pl.ds(h * D, D), pl.ds(q * TASK, TASK)],
            wsem,
        )
        return 0

    lax.fori_loop(0, n_tasks_per_w, step, 0)

    pltpu.make_async_copy(W_hbm.at[pl.ds(0, D)], tbuf_v.at[0], wsem).wait()
    pltpu.make_async_copy(W_hbm.at[pl.ds(0, D)], tbuf_v.at[1], wsem).wait()


@functools.partial(jax.jit, static_argnames=("n_hist", "n_batch"))
def _emb_lookup(x_t2d, W, n_hist, n_batch):
    n_rows = n_hist * n_batch
    n_tasks_per_w = n_rows // (NW * TASK)
    idx_rows = n_tasks_per_w * TASK_STREAMS
    mesh = plsc.VectorSubcoreMesh(core_axis_name="c", subcore_axis_name="s")
    run = pl.kernel(
        functools.partial(_emb_body, n_tasks_per_w, n_batch),
        out_type=jax.ShapeDtypeStruct((n_hist * D, n_batch), jnp.float32),
        mesh=mesh,
        scratch_types=[
            pltpu.VMEM((idx_rows, STREAM), jnp.int32),
            pltpu.VMEM((2, TASK, D), jnp.float32),
            pltpu.VMEM((2, D, TASK), jnp.float32),
            pltpu.SemaphoreType.DMA,
            pltpu.SemaphoreType.DMA,
        ],
        compiler_params=pltpu.CompilerParams(
            use_tc_tiling_on_sc=False, needs_layout_passes=False
        ),
    )
    return run(W, x_t2d)


def kernel(x, W):
    b, h = x.shape
    v = W.shape[0]
    x_t2d = x.T.reshape((h * b) // STREAM, STREAM)
    W_lin = _tc_transpose(W.T, v).reshape(-1, D)
    out = _emb_lookup(x_t2d, W_lin, h, b)
    return out.reshape(h, D, b).transpose(2, 0, 1)
